# speedup vs baseline: 4.8568x; 4.8568x over previous
import jax
import jax.numpy as jnp
from jax.experimental import pallas as pl
from jax.experimental.pallas import tpu as pltpu

_BLOCK = 2048
_TPAD = 128


def _body(pos_ref, x_ref, tab_ref, o_ref):
    idx = pos_ref[0, 0, :]
    cols = jax.lax.broadcasted_iota(jnp.int32, (1, _TPAD), 1)
    onehot = (idx[:, None] == cols).astype(jnp.float32)
    g = jax.lax.dot_general(
        onehot, tab_ref[...], (((1,), (0,)), ((), ())),
        preferred_element_type=jnp.float32)
    o_ref[...] = x_ref[...] + g


def kernel(inputs, inputs_positions, position_emb):
    B, N, D = inputs.shape
    tot = B * N
    nb = tot // _BLOCK
    x = inputs.reshape(tot, D)
    pos = inputs_positions.reshape(nb, 1, _BLOCK).astype(jnp.int32)
    table = jnp.squeeze(position_emb, axis=0)
    table = jnp.pad(table, ((0, _TPAD - table.shape[0]), (0, 0)))

    out = pl.pallas_call(
        _body,
        grid=(nb,),
        in_specs=[
            pl.BlockSpec((1, 1, _BLOCK), lambda i: (i, 0, 0)),
            pl.BlockSpec((_BLOCK, D), lambda i: (i, 0)),
            pl.BlockSpec((_TPAD, D), lambda i: (0, 0)),
        ],
        out_specs=pl.BlockSpec((_BLOCK, D), lambda i: (i, 0)),
        out_shape=jax.ShapeDtypeStruct((tot, D), jnp.float32),
    )(pos, x, table)
    return out.reshape(B, N, D)


# speedup vs baseline: 5.1137x; 1.0529x over previous
import jax
import jax.numpy as jnp
from jax.experimental import pallas as pl
from jax.experimental.pallas import tpu as pltpu

_BLOCK = 4096
_TPAD = 128


def _body(pos_ref, x_ref, tab_ref, o_ref):
    idx = pos_ref[0, 0, :]
    cols = jax.lax.broadcasted_iota(jnp.int32, (1, _TPAD), 1)
    onehot = (idx[:, None] == cols).astype(jnp.float32)
    g = jax.lax.dot_general(
        onehot, tab_ref[...], (((1,), (0,)), ((), ())),
        preferred_element_type=jnp.float32)
    o_ref[...] = x_ref[...] + g


def kernel(inputs, inputs_positions, position_emb):
    B, N, D = inputs.shape
    tot = B * N
    nb = tot // _BLOCK
    x = inputs.reshape(tot, D)
    pos = inputs_positions.reshape(nb, 1, _BLOCK).astype(jnp.int32)
    table = jnp.squeeze(position_emb, axis=0)
    table = jnp.pad(table, ((0, _TPAD - table.shape[0]), (0, 0)))

    out = pl.pallas_call(
        _body,
        grid=(nb,),
        in_specs=[
            pl.BlockSpec((1, 1, _BLOCK), lambda i: (i, 0, 0)),
            pl.BlockSpec((_BLOCK, D), lambda i: (i, 0)),
            pl.BlockSpec((_TPAD, D), lambda i: (0, 0)),
        ],
        out_specs=pl.BlockSpec((_BLOCK, D), lambda i: (i, 0)),
        out_shape=jax.ShapeDtypeStruct((tot, D), jnp.float32),
    )(pos, x, table)
    return out.reshape(B, N, D)


# speedup vs baseline: 5.2493x; 1.0265x over previous
import jax
import jax.numpy as jnp
from jax.experimental import pallas as pl
from jax.experimental.pallas import tpu as pltpu

_BLOCK = 8192
_TPAD = 128


def _body(pos_ref, x_ref, tab_ref, o_ref):
    idx = pos_ref[0, 0, :]
    cols = jax.lax.broadcasted_iota(jnp.int32, (1, _TPAD), 1)
    onehot = (idx[:, None] == cols).astype(jnp.float32)
    g = jax.lax.dot_general(
        onehot, tab_ref[...], (((1,), (0,)), ((), ())),
        preferred_element_type=jnp.float32)
    o_ref[...] = x_ref[...] + g


def kernel(inputs, inputs_positions, position_emb):
    B, N, D = inputs.shape
    tot = B * N
    nb = tot // _BLOCK
    x = inputs.reshape(tot, D)
    pos = inputs_positions.reshape(nb, 1, _BLOCK).astype(jnp.int32)
    table = jnp.squeeze(position_emb, axis=0)
    table = jnp.pad(table, ((0, _TPAD - table.shape[0]), (0, 0)))

    out = pl.pallas_call(
        _body,
        grid=(nb,),
        in_specs=[
            pl.BlockSpec((1, 1, _BLOCK), lambda i: (i, 0, 0)),
            pl.BlockSpec((_BLOCK, D), lambda i: (i, 0)),
            pl.BlockSpec((_TPAD, D), lambda i: (0, 0)),
        ],
        out_specs=pl.BlockSpec((_BLOCK, D), lambda i: (i, 0)),
        out_shape=jax.ShapeDtypeStruct((tot, D), jnp.float32),
    )(pos, x, table)
    return out.reshape(B, N, D)
